# baseline (speedup 1.0000x reference)
import functools

import jax
import jax.numpy as jnp
import numpy as np
from jax import lax
from jax.experimental import pallas as pl
from jax.experimental.pallas import tpu as pltpu
from jax.experimental.pallas import tpu_sc as plsc

D = 128
BATCH = 4096
SEQ = 50
NTOK = BATCH * SEQ
NC, NS, LANES = 2, 16, 16
NW = NC * NS
COMBO_PAD = 10240
TOK_PER_W = NTOK // NW
CHUNK = 256
NCHUNK = TOK_PER_W // CHUNK
EPS = 1e-5

XROWS = NTOK * 4 // 128
XBLK = 800

_QW = np.zeros((128, 32), np.float32)
for _i in range(128):
    _QW[_i, _i // 4] = float(10 ** (3 - _i % 4))


def _combo_ids_body(x_ref, w_ref, o_ref):
    xf = x_ref[...].astype(jnp.float32)
    o_ref[...] = jax.lax.dot(
        xf, w_ref[...], precision=jax.lax.Precision.HIGHEST).astype(jnp.int32)


_combo_ids_call = pl.pallas_call(
    _combo_ids_body,
    grid=(XROWS // XBLK,),
    in_specs=[
        pl.BlockSpec((XBLK, 128), lambda i: (i, 0)),
        pl.BlockSpec((128, 32), lambda i: (0, 0)),
    ],
    out_specs=pl.BlockSpec((XBLK, 32), lambda i: (i, 0)),
    out_shape=jax.ShapeDtypeStruct((XROWS, 32), jnp.int32),
)


def _combo_ids(xr):
    return _combo_ids_call(xr, jnp.asarray(_QW))

TBLK = 1024


def _build_table_body(tw_ref, cw_ref, nw_ref, dw_ref, g_ref, b_ref, o_ref):
    i = pl.program_id(0)
    r = lax.broadcasted_iota(jnp.int32, (TBLK, 1), 0) + i * TBLK
    cols = lax.broadcasted_iota(jnp.int32, (TBLK, 16), 1)

    def onehot(idx):
        return (idx == cols).astype(jnp.float32)

    hp = jax.lax.Precision.HIGHEST
    emb = (
        jax.lax.dot(onehot(r // 1000), tw_ref[...], precision=hp)
        + jax.lax.dot(onehot((r // 100) % 10), cw_ref[...], precision=hp)
        + jax.lax.dot(onehot((r // 10) % 10), nw_ref[...], precision=hp)
        + jax.lax.dot(onehot(r % 10), dw_ref[...], precision=hp)
    )
    mean = jnp.mean(emb, axis=1, keepdims=True)
    cent = emb - mean
    var = jnp.mean(cent * cent, axis=1, keepdims=True)
    xhat = cent * lax.rsqrt(var + EPS)
    o_ref[...] = xhat * g_ref[...] + b_ref[...]


_build_table = pl.pallas_call(
    _build_table_body,
    grid=(COMBO_PAD // TBLK,),
    in_specs=[
        pl.BlockSpec((16, D), lambda i: (0, 0)),
        pl.BlockSpec((16, D), lambda i: (0, 0)),
        pl.BlockSpec((16, D), lambda i: (0, 0)),
        pl.BlockSpec((16, D), lambda i: (0, 0)),
        pl.BlockSpec((1, D), lambda i: (0, 0)),
        pl.BlockSpec((1, D), lambda i: (0, 0)),
    ],
    out_specs=pl.BlockSpec((TBLK, D), lambda i: (i, 0)),
    out_shape=jax.ShapeDtypeStruct((COMBO_PAD, D), jnp.float32),
)


def _mesh():
    return plsc.VectorSubcoreMesh(
        core_axis_name="c", subcore_axis_name="s",
        num_cores=NC, num_subcores=NS)


@functools.partial(
    pl.kernel,
    out_type=jax.ShapeDtypeStruct((NTOK, D), jnp.float32),
    mesh=_mesh(),
    scratch_types=[
        pltpu.VMEM((CHUNK,), jnp.int32),
        pltpu.VMEM((CHUNK,), jnp.int32),
        pltpu.VMEM((CHUNK, D), jnp.float32),
        pltpu.VMEM((CHUNK, D), jnp.float32),
        pltpu.SemaphoreType.DMA,
        pltpu.SemaphoreType.DMA,
        pltpu.SemaphoreType.DMA,
        pltpu.SemaphoreType.DMA,
        pltpu.SemaphoreType.DMA,
        pltpu.SemaphoreType.DMA,
    ],
)
def _lookup(table_h, cid_h, out_h,
            combo_a, combo_b, rows_a, rows_b,
            sem_ga, sem_gb, sem_oa, sem_ob, sem_ca, sem_cb):
    wid = lax.axis_index("s") * NC + lax.axis_index("c")
    tok0 = wid * TOK_PER_W
    combos = (combo_a, combo_b)
    rows = (rows_a, rows_b)
    sem_g = (sem_ga, sem_gb)
    sem_o = (sem_oa, sem_ob)
    sem_c = (sem_ca, sem_cb)

    def cid_copy(t0, p):
        return pltpu.make_async_copy(
            cid_h.at[pl.ds(t0, CHUNK)], combos[p], sem_c[p])

    def gather_copies(p):
        c0 = pltpu.make_async_copy(table_h.at[combos[p].at[pl.ds(0, 128)]],
                                   rows[p].at[pl.ds(0, 128)], sem_g[p])
        c1 = pltpu.make_async_copy(table_h.at[combos[p].at[pl.ds(128, 128)]],
                                   rows[p].at[pl.ds(128, 128)], sem_g[p])
        return c0, c1

    def out_copy(t0, p):
        return pltpu.make_async_copy(rows[p], out_h.at[pl.ds(t0, CHUNK)],
                                     sem_o[p])

    def stage(t0, p, next_t0):
        for c in gather_copies(p):
            c.wait()
        out_copy(t0, p).start()
        if next_t0 is not None:
            cid_copy(next_t0, p).start()
            cid_copy(next_t0, p).wait()
        out_copy(t0, p).wait()
        if next_t0 is not None:
            for c in gather_copies(p):
                c.start()

    cid_copy(tok0, 0).start()
    cid_copy(tok0, 0).wait()
    for c in gather_copies(0):
        c.start()
    cid_copy(tok0 + CHUNK, 1).start()
    cid_copy(tok0 + CHUNK, 1).wait()
    for c in gather_copies(1):
        c.start()

    def body(k, carry):
        t0 = tok0 + (2 * k) * CHUNK
        stage(t0, 0, t0 + 2 * CHUNK)
        stage(t0 + CHUNK, 1, t0 + 3 * CHUNK)
        return carry

    lax.fori_loop(0, (NCHUNK - 5) // 2, body, 0)
    t20 = tok0 + (NCHUNK - 5) * CHUNK
    stage(t20, 0, t20 + 2 * CHUNK)
    stage(t20 + CHUNK, 1, t20 + 3 * CHUNK)
    stage(t20 + 2 * CHUNK, 0, t20 + 4 * CHUNK)
    stage(t20 + 3 * CHUNK, 1, None)
    stage(t20 + 4 * CHUNK, 0, None)


def _pad16(w):
    return jnp.zeros((16, D), jnp.float32).at[: w.shape[0]].set(w)


def kernel(x, type_w, color_w, num_w, dup_w, ln_g, ln_b):
    table = _build_table(
        _pad16(type_w), _pad16(color_w), _pad16(num_w), _pad16(dup_w),
        ln_g.reshape(1, D), ln_b.reshape(1, D))
    cids = _combo_ids(x.reshape(XROWS, 128)).reshape(NTOK)
    out = _lookup(table, cids)
    return out.reshape(BATCH, SEQ, D)
